# baseline (speedup 1.0000x reference)
import functools

import jax
import jax.numpy as jnp
from jax import lax
from jax.experimental import pallas as pl
from jax.experimental.pallas import tpu as pltpu
from jax.experimental.pallas import tpu_sc as plsc

ROWS = 16384
COLS = 1000
K = 200
L = 16
NC = 2
NS = 16
NW = NC * NS
ROWS_PER_W = ROWS // NW
C = 32
N_CHUNKS = ROWS_PER_W // C
GROUP_OFFS = tuple(range(0, K - L + 1, L)) + (K - L,)


def _xmask_kernel(x_hbm, mask_hbm, out_hbm, mask_v, in_v, out_v):
    wid = lax.axis_index("s") * NC + lax.axis_index("c")
    row0 = wid * ROWS_PER_W

    pltpu.sync_copy(mask_hbm, mask_v)
    mask_vecs = [mask_v[pl.ds(off, L)] for off in GROUP_OFFS]

    def chunk_body(g, carry):
        base = row0 + g * C
        pltpu.sync_copy(x_hbm.at[pl.ds(base, C), :], in_v)

        def row_body(r, c2):
            row_splat = jnp.full((L,), r, dtype=jnp.int32)
            for off, mvec in zip(GROUP_OFFS, mask_vecs):
                vals = plsc.load_gather(in_v, [row_splat, mvec])
                out_v[r, pl.ds(off, L)] = vals
            return c2

        lax.fori_loop(0, C, row_body, 0, unroll=4)
        pltpu.sync_copy(out_v, out_hbm.at[pl.ds(base, C), :])
        return carry

    lax.fori_loop(0, N_CHUNKS, chunk_body, 0)


def kernel(x, mask):
    mesh = plsc.VectorSubcoreMesh(core_axis_name="c", subcore_axis_name="s")
    run = pl.kernel(
        _xmask_kernel,
        mesh=mesh,
        out_type=jax.ShapeDtypeStruct((ROWS, K), jnp.float32),
        scratch_types=[
            pltpu.VMEM((K,), jnp.int32),
            pltpu.VMEM((C, COLS), jnp.float32),
            pltpu.VMEM((C, K), jnp.float32),
        ],
        compiler_params=pltpu.CompilerParams(
            use_tc_tiling_on_sc=False, needs_layout_passes=False
        ),
    )
    return run(x, mask)


# speedup vs baseline: 1.6290x; 1.6290x over previous
import functools

import jax
import jax.numpy as jnp
from jax import lax
from jax.experimental import pallas as pl
from jax.experimental.pallas import tpu as pltpu
from jax.experimental.pallas import tpu_sc as plsc

ROWS = 16384
COLS = 1000
K = 200
L = 16
NC = 2
NS = 16
NW = NC * NS
ROWS_PER_W = ROWS // NW
C = 32
N_CHUNKS = ROWS_PER_W // C
GROUP_OFFS = tuple(range(0, K - L + 1, L)) + (K - L,)


def _xmask_kernel(x_hbm, mask_hbm, out_hbm, mask_v, in_v, out_v):
    wid = lax.axis_index("s") * NC + lax.axis_index("c")
    row0 = wid * ROWS_PER_W

    pltpu.sync_copy(mask_hbm, mask_v)
    mask_vecs = [mask_v[pl.ds(off, L)] for off in GROUP_OFFS]

    def chunk_body(g, carry):
        base = row0 + g * C
        pltpu.sync_copy(x_hbm.at[pl.ds(base, C), :], in_v)

        def row_body(r, c2):
            row_splat = jnp.full((L,), r, dtype=jnp.int32)
            for off, mvec in zip(GROUP_OFFS, mask_vecs):
                vals = plsc.load_gather(in_v, [row_splat, mvec])
                out_v[r, pl.ds(off, L)] = vals
            return c2

        lax.fori_loop(0, C, row_body, 0, unroll=4)
        pltpu.sync_copy(out_v, out_hbm.at[pl.ds(base, C), :])
        return carry

    lax.fori_loop(0, N_CHUNKS, chunk_body, 0)


def kernel(x, mask):
    mesh = plsc.VectorSubcoreMesh(core_axis_name="c", subcore_axis_name="s")
    run = pl.kernel(
        _xmask_kernel,
        mesh=mesh,
        out_type=jax.ShapeDtypeStruct((ROWS, K), jnp.float32),
        scratch_types=[
            pltpu.VMEM((K,), jnp.int32),
            pltpu.VMEM((C, COLS), jnp.float32),
            pltpu.VMEM((C, K), jnp.float32),
        ],
        compiler_params=pltpu.CompilerParams(needs_layout_passes=False),
    )
    return run(x, mask)


# speedup vs baseline: 7.6440x; 4.6925x over previous
import jax
import jax.numpy as jnp
from jax import lax
from jax.experimental import pallas as pl
from jax.experimental.pallas import tpu as pltpu
from jax.experimental.pallas import tpu_sc as plsc

ROWS = 16384
COLS = 1000
K = 200
L = 16
NW = 32
NGROUPS = 13
CBLK = 4096
NCB = ROWS // CBLK
NTASKS = NGROUPS * NCB
MAX_TASKS_PER_W = (NTASKS + NW - 1) // NW


def _xmask_kernel(xt_hbm, mask_hbm, out_hbm, mask_v, buf_v, sem):
    wid = lax.axis_index("s") * 2 + lax.axis_index("c")
    pltpu.sync_copy(mask_hbm, mask_v)

    for k in range(MAX_TASKS_PER_W):
        t = wid + NW * k
        g = t // NCB
        cb = t % NCB
        off = jnp.where(g < NGROUPS - 1, g * L, K - L)
        c0 = cb * CBLK

        @pl.when(t < NTASKS)
        def _task():
            idx = mask_v[pl.ds(off, L)]
            pltpu.async_copy(
                xt_hbm.at[idx, pl.ds(c0, CBLK)], buf_v, sem
            ).wait()
            pltpu.sync_copy(buf_v, out_hbm.at[pl.ds(off, L), pl.ds(c0, CBLK)])

    return


def kernel(x, mask):
    mesh = plsc.VectorSubcoreMesh(core_axis_name="c", subcore_axis_name="s")
    run = pl.kernel(
        _xmask_kernel,
        mesh=mesh,
        out_type=jax.ShapeDtypeStruct((K, ROWS), jnp.float32),
        scratch_types=[
            pltpu.VMEM((K,), jnp.int32),
            pltpu.VMEM((L, CBLK), jnp.float32),
            pltpu.SemaphoreType.DMA,
        ],
        compiler_params=pltpu.CompilerParams(needs_layout_passes=False),
    )
    return run(x.T, mask).T


# speedup vs baseline: 7.8483x; 1.0267x over previous
import jax
import jax.numpy as jnp
from jax import lax
from jax.experimental import pallas as pl
from jax.experimental.pallas import tpu as pltpu
from jax.experimental.pallas import tpu_sc as plsc

ROWS = 16384
COLS = 1000
K = 200
L = 16
NW = 32
NGROUPS = 13
CBLK = 2048
NCB = ROWS // CBLK
NTASKS = NGROUPS * NCB
MAX_TASKS_PER_W = (NTASKS + NW - 1) // NW


def _task_coords(t):
    g = t // NCB
    cb = t % NCB
    off = jnp.where(g < NGROUPS - 1, g * L, K - L)
    return off, cb * CBLK


def _xmask_kernel(xt_hbm, mask_hbm, out_hbm, mask_v, buf0, buf1, sem0, sem1):
    wid = lax.axis_index("s") * 2 + lax.axis_index("c")
    pltpu.sync_copy(mask_hbm, mask_v)
    bufs = (buf0, buf1)
    sems = (sem0, sem1)

    def start_gather(t, b):
        @pl.when(t < NTASKS)
        def _():
            off, c0 = _task_coords(t)
            idx = mask_v[pl.ds(off, L)]
            pltpu.make_async_copy(
                xt_hbm.at[idx, pl.ds(c0, CBLK)], bufs[b], sems[b]
            ).start()

    def drain_and_write(t, b):
        @pl.when(t < NTASKS)
        def _():
            off, c0 = _task_coords(t)
            idx = mask_v[pl.ds(off, L)]
            pltpu.make_async_copy(
                xt_hbm.at[idx, pl.ds(c0, CBLK)], bufs[b], sems[b]
            ).wait()
            pltpu.sync_copy(
                bufs[b], out_hbm.at[pl.ds(off, L), pl.ds(c0, CBLK)]
            )

    start_gather(wid, 0)
    for k in range(MAX_TASKS_PER_W):
        if k + 1 < MAX_TASKS_PER_W:
            start_gather(wid + NW * (k + 1), (k + 1) % 2)
        drain_and_write(wid + NW * k, k % 2)

    return


def kernel(x, mask):
    mesh = plsc.VectorSubcoreMesh(core_axis_name="c", subcore_axis_name="s")
    run = pl.kernel(
        _xmask_kernel,
        mesh=mesh,
        out_type=jax.ShapeDtypeStruct((K, ROWS), jnp.float32),
        scratch_types=[
            pltpu.VMEM((K,), jnp.int32),
            pltpu.VMEM((L, CBLK), jnp.float32),
            pltpu.VMEM((L, CBLK), jnp.float32),
            pltpu.SemaphoreType.DMA,
            pltpu.SemaphoreType.DMA,
        ],
        compiler_params=pltpu.CompilerParams(needs_layout_passes=False),
    )
    return run(x.T, mask).T
